# Initial kernel scaffold; baseline (speedup 1.0000x reference)
import jax
import jax.numpy as jnp
from jax.experimental import pallas as pl


def kernel(seq_len, embedding):
    raise NotImplementedError("write your pallas kernel here")


# speedup vs baseline: 1.5614x; 1.5614x over previous
import functools

import jax
import jax.numpy as jnp
from jax import lax
from jax.experimental import pallas as pl
from jax.experimental.pallas import tpu as pltpu
from jax.experimental.pallas import tpu_sc as plsc

MAX_LEN = 8192
DIM = 2048
NC = 2
NS = 16
NW = NC * NS
ROWS_PER_W = MAX_LEN // NW
CHUNK = 16
NCHUNK = ROWS_PER_W // CHUNK
NBUF = 2


def _lookup_body(clamp_hbm, table_hbm, out_hbm, clamp_v, buf0, buf1,
                 gsem, wsem):
    wid = lax.axis_index("s") * NC + lax.axis_index("c")
    base = wid * ROWS_PER_W

    pltpu.sync_copy(clamp_hbm, clamp_v)
    clamp = clamp_v[...]
    iota = lax.iota(jnp.int32, CHUNK)
    bufs = (buf0, buf1)

    def idx_for(g):
        return jnp.minimum(base + g * CHUNK + iota, clamp)

    gh = [None] * NCHUNK
    wh = [None] * NCHUNK
    for g in range(NCHUNK):
        b = g % NBUF
        if g >= NBUF:
            wh[g - NBUF].wait()
        gh[g] = pltpu.async_copy(table_hbm.at[idx_for(g)], bufs[b], gsem)
        if g >= 1:
            p = g - 1
            gh[p].wait()
            wh[p] = pltpu.async_copy(
                bufs[p % NBUF],
                out_hbm.at[pl.ds(base + p * CHUNK, CHUNK)],
                wsem,
            )
    last = NCHUNK - 1
    gh[last].wait()
    wh[last] = pltpu.async_copy(
        bufs[last % NBUF],
        out_hbm.at[pl.ds(base + last * CHUNK, CHUNK)],
        wsem,
    )
    wh[last - 1].wait()
    wh[last].wait()


_lookup = functools.partial(
    pl.kernel,
    out_type=jax.ShapeDtypeStruct((MAX_LEN, DIM), jnp.float32),
    mesh=plsc.VectorSubcoreMesh(core_axis_name="c", subcore_axis_name="s"),
    scratch_types=[
        pltpu.VMEM((CHUNK,), jnp.int32),
        pltpu.VMEM((CHUNK, DIM), jnp.float32),
        pltpu.VMEM((CHUNK, DIM), jnp.float32),
        pltpu.SemaphoreType.DMA,
        pltpu.SemaphoreType.DMA,
    ],
)(_lookup_body)


def kernel(seq_len, embedding):
    clamp = jnp.full((CHUNK,), seq_len - 1, dtype=jnp.int32)
    return _lookup(clamp, embedding)


# speedup vs baseline: 1.5826x; 1.0136x over previous
import functools

import jax
import jax.numpy as jnp
from jax import lax
from jax.experimental import pallas as pl
from jax.experimental.pallas import tpu as pltpu
from jax.experimental.pallas import tpu_sc as plsc

MAX_LEN = 8192
DIM = 2048
NC = 2
NS = 16
NW = NC * NS
ROWS_PER_W = MAX_LEN // NW
CHUNK = 16
NCHUNK = ROWS_PER_W // CHUNK
NBUF = 3


def _lookup_body(clamp_hbm, table_hbm, out_hbm, clamp_v, buf0, buf1, buf2,
                 gsem, wsem):
    wid = lax.axis_index("s") * NC + lax.axis_index("c")
    base = wid * ROWS_PER_W

    pltpu.sync_copy(clamp_hbm, clamp_v)
    clamp = clamp_v[...]
    iota = lax.iota(jnp.int32, CHUNK)
    bufs = (buf0, buf1, buf2)

    def idx_for(g):
        return jnp.minimum(base + g * CHUNK + iota, clamp)

    gh = [None] * NCHUNK
    wh = [None] * NCHUNK
    for g in range(NCHUNK):
        b = g % NBUF
        if g >= NBUF:
            wh[g - NBUF].wait()
        gh[g] = pltpu.async_copy(table_hbm.at[idx_for(g)], bufs[b], gsem)
        if g >= 1:
            p = g - 1
            gh[p].wait()
            wh[p] = pltpu.async_copy(
                bufs[p % NBUF],
                out_hbm.at[pl.ds(base + p * CHUNK, CHUNK)],
                wsem,
            )
    last = NCHUNK - 1
    gh[last].wait()
    wh[last] = pltpu.async_copy(
        bufs[last % NBUF],
        out_hbm.at[pl.ds(base + last * CHUNK, CHUNK)],
        wsem,
    )
    wh[last - 1].wait()
    wh[last].wait()


_lookup = functools.partial(
    pl.kernel,
    out_type=jax.ShapeDtypeStruct((MAX_LEN, DIM), jnp.float32),
    mesh=plsc.VectorSubcoreMesh(core_axis_name="c", subcore_axis_name="s"),
    scratch_types=[
        pltpu.VMEM((CHUNK,), jnp.int32),
        pltpu.VMEM((CHUNK, DIM), jnp.float32),
        pltpu.VMEM((CHUNK, DIM), jnp.float32),
        pltpu.VMEM((CHUNK, DIM), jnp.float32),
        pltpu.SemaphoreType.DMA,
        pltpu.SemaphoreType.DMA,
    ],
)(_lookup_body)


def kernel(seq_len, embedding):
    clamp = jnp.full((CHUNK,), seq_len - 1, dtype=jnp.int32)
    return _lookup(clamp, embedding)


# speedup vs baseline: 1.5878x; 1.0033x over previous
import functools

import jax
import jax.numpy as jnp
from jax import lax
from jax.experimental import pallas as pl
from jax.experimental.pallas import tpu as pltpu
from jax.experimental.pallas import tpu_sc as plsc

MAX_LEN = 8192
DIM = 2048
NC = 2
NS = 16
NW = NC * NS
ROWS_PER_W = MAX_LEN // NW
CHUNK = 16
NCHUNK = ROWS_PER_W // CHUNK
NBUF = 3


def _lookup_body(clamp_hbm, table_hbm, out_hbm, clamp_v, buf0, buf1, buf2,
                 gsem, wsem):
    wid = lax.axis_index("s") * NC + lax.axis_index("c")
    base = wid * ROWS_PER_W

    pltpu.sync_copy(clamp_hbm, clamp_v)
    clamp = clamp_v[...]
    iota = lax.iota(jnp.int32, CHUNK)
    bufs = (buf0, buf1, buf2)

    def idx_for(g):
        return jnp.minimum(base + g * CHUNK + iota, clamp)

    gh = [None] * NCHUNK
    wh = [None] * NCHUNK
    for g in range(NCHUNK):
        b = g % NBUF
        if g >= NBUF:
            wh[g - NBUF].wait()
        gh[g] = pltpu.async_copy(table_hbm.at[idx_for(g)], bufs[b], gsem)
        if g >= 1:
            p = g - 1
            gh[p].wait()
            wh[p] = pltpu.async_copy(
                bufs[p % NBUF],
                out_hbm.at[pl.ds(base + p * CHUNK, CHUNK)],
                wsem,
            )
    last = NCHUNK - 1
    gh[last].wait()
    wh[last] = pltpu.async_copy(
        bufs[last % NBUF],
        out_hbm.at[pl.ds(base + last * CHUNK, CHUNK)],
        wsem,
    )
    for p in range(NCHUNK - NBUF, NCHUNK):
        wh[p].wait()


_lookup = functools.partial(
    pl.kernel,
    out_type=jax.ShapeDtypeStruct((MAX_LEN, DIM), jnp.float32),
    mesh=plsc.VectorSubcoreMesh(core_axis_name="c", subcore_axis_name="s"),
    scratch_types=[
        pltpu.VMEM((CHUNK,), jnp.int32),
        pltpu.VMEM((CHUNK, DIM), jnp.float32),
        pltpu.VMEM((CHUNK, DIM), jnp.float32),
        pltpu.VMEM((CHUNK, DIM), jnp.float32),
        pltpu.SemaphoreType.DMA,
        pltpu.SemaphoreType.DMA,
    ],
)(_lookup_body)


def kernel(seq_len, embedding):
    clamp = jnp.full((CHUNK,), seq_len - 1, dtype=jnp.int32)
    return _lookup(clamp, embedding)
